# Initial kernel scaffold; baseline (speedup 1.0000x reference)
import jax
import jax.numpy as jnp
from jax.experimental import pallas as pl


def kernel(x, target):
    raise NotImplementedError("write your pallas kernel here")


# speedup vs baseline: 1.4656x; 1.4656x over previous
import functools

import jax
import jax.numpy as jnp
from jax.experimental import pallas as pl

_EPS = 1e-7
_IGNORE = 0.7
_FM0 = 76
_CELLS_OBJ = _FM0 * _FM0
_C = 85
_WIDE = 3 * _C
_ANCHORS = ((10.0, 13.0), (16.0, 30.0), (33.0, 23.0))

_ROWS_PER_BLK = 1083
_NBLK = 7


def _loss_kernel(x_ref, t_ref, out_ref):
    b = pl.program_id(0)
    j = pl.program_id(1)

    xv = x_ref[0, 0]
    tv = t_ref[0, 0]

    one = jnp.float32(1.0)
    eps = jnp.float32(_EPS)
    p = jnp.clip(xv, eps, one - eps)
    bce = -(tv * jnp.log(p) + (one - tv) * jnp.log(one - p))

    lane = jax.lax.broadcasted_iota(jnp.int32, (1, _WIDE), 1)

    def lanes(*cols):
        m = (lane == cols[0])
        for c in cols[1:]:
            m = m | (lane == c)
        return m

    is_xy = lanes(0, 1, 85, 86, 170, 171)
    is_wh = lanes(2, 3, 87, 88, 172, 173)
    is_cls = lanes(4, 89, 174)

    zero = jnp.float32(0.0)
    m0 = jnp.where(tv[:, 4:5] > zero, one, zero)
    m1 = jnp.where(tv[:, 89:90] > zero, one, zero)
    m2 = jnp.where(tv[:, 174:175] > zero, one, zero)
    m_full = jnp.where(lane < _C, m0, jnp.where(lane < 2 * _C, m1, m2))

    bm = bce * m_full
    s_xy = jnp.sum(jnp.where(is_xy, bm, zero))
    s_wh = jnp.sum(jnp.where(is_wh, bm, zero))
    s_cls = jnp.sum(jnp.where(is_cls, bm, zero))
    s_m = jnp.sum(m0) + jnp.sum(m1) + jnp.sum(m2)

    def iou_k(k):
        aw, ah = _ANCHORS[k]
        w = tv[:, k * _C + 2:k * _C + 3]
        h = tv[:, k * _C + 3:k * _C + 4]
        inter = jnp.minimum(jnp.float32(aw), w) * jnp.minimum(jnp.float32(ah), h)
        return inter / (jnp.float32(aw * ah) + w * h - inter + jnp.float32(1e-16))

    iou0, iou1, iou2 = iou_k(0), iou_k(1), iou_k(2)
    best0 = (iou0 >= iou1) & (iou0 >= iou2)
    best1 = jnp.logical_not(best0) & (iou1 >= iou2)
    best2 = jnp.logical_not(best0 | best1)
    ig = jnp.float32(_IGNORE)
    mk0 = jnp.where(best0 | (iou0 <= ig), one, zero)
    mk1 = jnp.where(best1 | (iou1 <= ig), one, zero)
    mk2 = jnp.where(best2 | (iou2 <= ig), one, zero)
    mask_full = jnp.where(lane < _C, mk0, jnp.where(lane < 2 * _C, mk1, mk2))

    c0 = -jnp.log(one - eps)
    obj_elem = c0 + mask_full * (bce - c0)

    row = jax.lax.broadcasted_iota(jnp.int32, (_ROWS_PER_BLK, 1), 0)
    is_obj = jnp.where((j * _ROWS_PER_BLK + row) < _CELLS_OBJ, one, zero)
    s_obj = jnp.sum(obj_elem * is_obj)

    acc_lane = jax.lax.broadcasted_iota(jnp.int32, (8, 128), 1)
    partial = (
        jnp.where(acc_lane == 0, s_xy, zero)
        + jnp.where(acc_lane == 1, s_wh, zero)
        + jnp.where(acc_lane == 2, s_cls, zero)
        + jnp.where(acc_lane == 3, s_m, zero)
        + jnp.where(acc_lane == 4, s_obj, zero)
    )

    @pl.when((b == 0) & (j == 0))
    def _init():
        out_ref[...] = jnp.zeros_like(out_ref)

    out_ref[...] += partial


@jax.jit
def kernel(x, target):
    B, N, C = x.shape
    cells = N // 3
    xw = x.reshape(B, _NBLK, _ROWS_PER_BLK, _WIDE)
    tw = target.reshape(B, _NBLK, _ROWS_PER_BLK, _WIDE)

    out = pl.pallas_call(
        _loss_kernel,
        grid=(B, _NBLK),
        in_specs=[
            pl.BlockSpec((1, 1, _ROWS_PER_BLK, _WIDE), lambda b, j: (b, j, 0, 0)),
            pl.BlockSpec((1, 1, _ROWS_PER_BLK, _WIDE), lambda b, j: (b, j, 0, 0)),
        ],
        out_specs=pl.BlockSpec((8, 128), lambda b, j: (0, 0)),
        out_shape=jax.ShapeDtypeStruct((8, 128), jnp.float32),
    )(xw, tw)

    s_xy = out[0, 0]
    s_wh = out[0, 1]
    s_cls = out[0, 2]
    s_m = out[0, 3]
    s_obj = out[0, 4]

    n_obj = jnp.float32(B * _CELLS_OBJ * 3 * _C)
    return (s_xy + s_wh) / (2.0 * s_m) + s_cls / s_m + s_obj / n_obj


# speedup vs baseline: 1.8637x; 1.2716x over previous
import jax
import jax.numpy as jnp
from jax.experimental import pallas as pl
from jax.experimental.pallas import tpu as pltpu

_EPS = 1e-7
_IGNORE = 0.7
_N = 22743
_N_OBJ = 17328
_C = 85
_ANCHORS = ((10.0, 13.0), (16.0, 30.0), (33.0, 23.0))

_BR = 912
_JOBJ = _N_OBJ // _BR
_NJ = -(-_N // _BR)


def _loss_kernel(x_ref, t_ref, out_ref):
    b = pl.program_id(0)
    j = pl.program_id(1)

    xv = x_ref[0]
    tv = t_ref[0]

    one = jnp.float32(1.0)
    zero = jnp.float32(0.0)
    eps = jnp.float32(_EPS)
    p = jnp.clip(xv, eps, one - eps)
    log1mp = jnp.log(one - p)
    d = jnp.log(p) - log1mp
    nbce = tv * d + log1mp

    row = jax.lax.broadcasted_iota(jnp.int32, (_BR, 1), 0)
    valid = (j * _BR + row) < _N
    m = jnp.where(tv[:, 4:5] > zero, one, zero)

    xyr = jnp.where(valid, nbce[:, 0:1] + nbce[:, 1:2], zero)
    whr = jnp.where(valid, nbce[:, 2:3] + nbce[:, 3:4], zero)
    clsr = jnp.where(valid, nbce[:, 4:5], zero)
    mv = jnp.where(valid, m, zero)

    s_xy = -jnp.sum(xyr * mv)
    s_wh = -jnp.sum(whr * mv)
    s_cls = -jnp.sum(clsr * mv)
    s_m = jnp.sum(mv)

    acc_lane = jax.lax.broadcasted_iota(jnp.int32, (8, 128), 1)

    @pl.when((b == 0) & (j == 0))
    def _init():
        out_ref[...] = jnp.zeros_like(out_ref)

    out_ref[...] += (
        jnp.where(acc_lane == 0, s_xy, zero)
        + jnp.where(acc_lane == 1, s_wh, zero)
        + jnp.where(acc_lane == 2, s_cls, zero)
        + jnp.where(acc_lane == 3, s_m, zero)
    )

    @pl.when(j < _JOBJ)
    def _obj():
        rf = row.astype(jnp.float32)
        kf = rf - 3.0 * jnp.floor(rf * (1.0 / 3.0) + 1e-4)
        k0 = kf < 0.5
        k1 = (kf >= 0.5) & (kf < 1.5)

        aw = jnp.where(k0, _ANCHORS[0][0], jnp.where(k1, _ANCHORS[1][0], _ANCHORS[2][0]))
        ah = jnp.where(k0, _ANCHORS[0][1], jnp.where(k1, _ANCHORS[1][1], _ANCHORS[2][1]))
        area = aw * ah
        w = tv[:, 2:3]
        h = tv[:, 3:4]
        inter = jnp.minimum(aw, w) * jnp.minimum(ah, h)
        iou = inter / (area + w * h - inter + jnp.float32(1e-16))

        prev1 = pltpu.roll(iou, 1, 0)
        prev2 = pltpu.roll(iou, 2, 0)
        next1 = pltpu.roll(iou, _BR - 1, 0)
        next2 = pltpu.roll(iou, _BR - 2, 0)
        ciou0 = jnp.where(k0, iou, jnp.where(k1, prev1, prev2))
        ciou1 = jnp.where(k0, next1, jnp.where(k1, iou, prev1))
        ciou2 = jnp.where(k0, next2, jnp.where(k1, next1, iou))
        b0 = (ciou0 >= ciou1) & (ciou0 >= ciou2)
        b1 = jnp.logical_not(b0) & (ciou1 >= ciou2)
        b2 = jnp.logical_not(b0 | b1)
        is_best = (k0 & b0) | (k1 & b1) | ((kf >= 1.5) & b2)
        maskr = jnp.where(is_best | (iou <= jnp.float32(_IGNORE)), one, zero)

        rb = jnp.sum(nbce, axis=1, keepdims=True)
        c0 = -jnp.log(one - eps)
        s_obj = -jnp.sum(maskr * rb) + jnp.float32(_C) * c0 * (
            jnp.float32(_BR) - jnp.sum(maskr))
        out_ref[...] += jnp.where(acc_lane == 4, s_obj, zero)


@jax.jit
def kernel(x, target):
    B = x.shape[0]

    out = pl.pallas_call(
        _loss_kernel,
        grid=(B, _NJ),
        in_specs=[
            pl.BlockSpec((1, _BR, _C), lambda b, j: (b, j, 0)),
            pl.BlockSpec((1, _BR, _C), lambda b, j: (b, j, 0)),
        ],
        out_specs=pl.BlockSpec((8, 128), lambda b, j: (0, 0)),
        out_shape=jax.ShapeDtypeStruct((8, 128), jnp.float32),
    )(x, target)

    s_xy = out[0, 0]
    s_wh = out[0, 1]
    s_cls = out[0, 2]
    s_m = out[0, 3]
    s_obj = out[0, 4]

    n_obj = jnp.float32(B * _N_OBJ * _C)
    return (s_xy + s_wh) / (2.0 * s_m) + s_cls / s_m + s_obj / n_obj


# speedup vs baseline: 3.6609x; 1.9643x over previous
import jax
import jax.numpy as jnp
from jax.experimental import pallas as pl
from jax.experimental.pallas import tpu as pltpu

_EPS = 1e-7
_IGNORE = 0.7
_N = 22743
_N_OBJ = 17328
_C = 85
_ANCHORS = ((10.0, 13.0), (16.0, 30.0), (33.0, 23.0))

_BR = 912
_JOBJ = _N_OBJ // _BR
_NJ = -(-_N // _BR)

_DN_T = (((1,), (1,)), ((), ()))


def _loss_kernel(x_ref, t_ref, out_ref):
    b = pl.program_id(0)
    j = pl.program_id(1)

    xv = x_ref[0]
    tv = t_ref[0]

    one = jnp.float32(1.0)
    zero = jnp.float32(0.0)
    eps = jnp.float32(_EPS)
    p = jnp.clip(xv, eps, one - eps)
    log1mp = jnp.log(one - p)
    nbce = tv * (jnp.log(p) - log1mp) + log1mp

    sub4 = jax.lax.broadcasted_iota(jnp.int32, (4, _C), 0)
    lan4 = jax.lax.broadcasted_iota(jnp.int32, (4, _C), 1)
    w4 = jnp.where(
        ((sub4 == 0) & (lan4 < 2))
        | ((sub4 == 1) & ((lan4 == 2) | (lan4 == 3)))
        | ((sub4 == 2) & (lan4 == 4))
        | (sub4 == 3),
        one, zero)
    sub3 = jax.lax.broadcasted_iota(jnp.int32, (3, _C), 0)
    lan3 = jax.lax.broadcasted_iota(jnp.int32, (3, _C), 1)
    w3 = jnp.where(lan3 == sub3 + 2, one, zero)

    r4 = jax.lax.dot_general(w4, nbce, _DN_T,
                             preferred_element_type=jnp.float32)
    t3 = jax.lax.dot_general(w3, tv, _DN_T,
                             preferred_element_type=jnp.float32)

    lane = jax.lax.broadcasted_iota(jnp.int32, (1, _BR), 1)
    validb = (j * _BR + lane) < _N
    mt = jnp.where(t3[2:3, :] > zero, one, zero)
    mtv = jnp.where(validb, mt, zero)

    s_xy = -jnp.sum(jnp.where(validb, r4[0:1, :] * mt, zero))
    s_wh = -jnp.sum(jnp.where(validb, r4[1:2, :] * mt, zero))
    s_cls = -jnp.sum(jnp.where(validb, r4[2:3, :] * mt, zero))
    s_m = jnp.sum(mtv)

    acc_lane = jax.lax.broadcasted_iota(jnp.int32, (8, 128), 1)

    @pl.when((b == 0) & (j == 0))
    def _init():
        out_ref[...] = jnp.zeros_like(out_ref)

    out_ref[...] += (
        jnp.where(acc_lane == 0, s_xy, zero)
        + jnp.where(acc_lane == 1, s_wh, zero)
        + jnp.where(acc_lane == 2, s_cls, zero)
        + jnp.where(acc_lane == 3, s_m, zero)
    )

    @pl.when(j < _JOBJ)
    def _obj():
        lf = lane.astype(jnp.float32)
        kf = lf - 3.0 * jnp.floor(lf * (1.0 / 3.0) + 1e-4)
        k0 = kf < 0.5
        k1 = (kf >= 0.5) & (kf < 1.5)

        aw = jnp.where(k0, _ANCHORS[0][0], jnp.where(k1, _ANCHORS[1][0], _ANCHORS[2][0]))
        ah = jnp.where(k0, _ANCHORS[0][1], jnp.where(k1, _ANCHORS[1][1], _ANCHORS[2][1]))
        area = aw * ah
        w = t3[0:1, :]
        h = t3[1:2, :]
        inter = jnp.minimum(aw, w) * jnp.minimum(ah, h)
        iou = inter / (area + w * h - inter + jnp.float32(1e-16))

        prev1 = pltpu.roll(iou, 1, 1)
        prev2 = pltpu.roll(iou, 2, 1)
        next1 = pltpu.roll(iou, _BR - 1, 1)
        next2 = pltpu.roll(iou, _BR - 2, 1)
        ciou0 = jnp.where(k0, iou, jnp.where(k1, prev1, prev2))
        ciou1 = jnp.where(k0, next1, jnp.where(k1, iou, prev1))
        ciou2 = jnp.where(k0, next2, jnp.where(k1, next1, iou))
        b0 = (ciou0 >= ciou1) & (ciou0 >= ciou2)
        b1 = jnp.logical_not(b0) & (ciou1 >= ciou2)
        b2 = jnp.logical_not(b0 | b1)
        is_best = (k0 & b0) | (k1 & b1) | ((kf >= 1.5) & b2)
        maskr = jnp.where(is_best | (iou <= jnp.float32(_IGNORE)), one, zero)

        c0 = -jnp.log(one - eps)
        s_obj = -jnp.sum(maskr * r4[3:4, :]) + jnp.float32(_C) * c0 * (
            jnp.float32(_BR) - jnp.sum(maskr))
        out_ref[...] += jnp.where(acc_lane == 4, s_obj, zero)


@jax.jit
def kernel(x, target):
    B = x.shape[0]

    out = pl.pallas_call(
        _loss_kernel,
        grid=(B, _NJ),
        in_specs=[
            pl.BlockSpec((1, _BR, _C), lambda b, j: (b, j, 0)),
            pl.BlockSpec((1, _BR, _C), lambda b, j: (b, j, 0)),
        ],
        out_specs=pl.BlockSpec((8, 128), lambda b, j: (0, 0)),
        out_shape=jax.ShapeDtypeStruct((8, 128), jnp.float32),
    )(x, target)

    s_xy = out[0, 0]
    s_wh = out[0, 1]
    s_cls = out[0, 2]
    s_m = out[0, 3]
    s_obj = out[0, 4]

    n_obj = jnp.float32(B * _N_OBJ * _C)
    return (s_xy + s_wh) / (2.0 * s_m) + s_cls / s_m + s_obj / n_obj


# speedup vs baseline: 5.2375x; 1.4307x over previous
import jax
import jax.numpy as jnp
from jax.experimental import pallas as pl
from jax.experimental.pallas import tpu as pltpu

_EPS = 1e-7
_IGNORE = 0.7
_N = 22743
_N_OBJ = 17328
_C = 85
_ANCHORS = ((10.0, 13.0), (16.0, 30.0), (33.0, 23.0))

_BR = 8664
_JOBJ = _N_OBJ // _BR
_NJ = -(-_N // _BR)

_DN_T = (((1,), (1,)), ((), ()))


def _loss_kernel(x_ref, t_ref, out_ref):
    b = pl.program_id(0)
    j = pl.program_id(1)

    xv = x_ref[0]
    tv = t_ref[0]

    one = jnp.float32(1.0)
    zero = jnp.float32(0.0)
    eps = jnp.float32(_EPS)
    p = jnp.clip(xv, eps, one - eps)
    log1mp = jnp.log(one - p)
    nbce = tv * (jnp.log(p) - log1mp) + log1mp

    sub4 = jax.lax.broadcasted_iota(jnp.int32, (4, _C), 0)
    lan4 = jax.lax.broadcasted_iota(jnp.int32, (4, _C), 1)
    w4 = jnp.where(
        ((sub4 == 0) & (lan4 < 2))
        | ((sub4 == 1) & ((lan4 == 2) | (lan4 == 3)))
        | ((sub4 == 2) & (lan4 == 4))
        | (sub4 == 3),
        one, zero)
    sub3 = jax.lax.broadcasted_iota(jnp.int32, (3, _C), 0)
    lan3 = jax.lax.broadcasted_iota(jnp.int32, (3, _C), 1)
    w3 = jnp.where(lan3 == sub3 + 2, one, zero)

    r4 = jax.lax.dot_general(w4, nbce, _DN_T,
                             preferred_element_type=jnp.float32)
    t3 = jax.lax.dot_general(w3, tv, _DN_T,
                             preferred_element_type=jnp.float32)

    lane = jax.lax.broadcasted_iota(jnp.int32, (1, _BR), 1)
    validb = (j * _BR + lane) < _N
    mt = jnp.where(t3[2:3, :] > zero, one, zero)
    mtv = jnp.where(validb, mt, zero)

    s_xy = -jnp.sum(jnp.where(validb, r4[0:1, :] * mt, zero))
    s_wh = -jnp.sum(jnp.where(validb, r4[1:2, :] * mt, zero))
    s_cls = -jnp.sum(jnp.where(validb, r4[2:3, :] * mt, zero))
    s_m = jnp.sum(mtv)

    acc_lane = jax.lax.broadcasted_iota(jnp.int32, (8, 128), 1)

    @pl.when((b == 0) & (j == 0))
    def _init():
        out_ref[...] = jnp.zeros_like(out_ref)

    out_ref[...] += (
        jnp.where(acc_lane == 0, s_xy, zero)
        + jnp.where(acc_lane == 1, s_wh, zero)
        + jnp.where(acc_lane == 2, s_cls, zero)
        + jnp.where(acc_lane == 3, s_m, zero)
    )

    @pl.when(j < _JOBJ)
    def _obj():
        lf = lane.astype(jnp.float32)
        kf = lf - 3.0 * jnp.floor(lf * (1.0 / 3.0) + 0.15)
        k0 = kf < 0.5
        k1 = (kf >= 0.5) & (kf < 1.5)

        aw = jnp.where(k0, _ANCHORS[0][0], jnp.where(k1, _ANCHORS[1][0], _ANCHORS[2][0]))
        ah = jnp.where(k0, _ANCHORS[0][1], jnp.where(k1, _ANCHORS[1][1], _ANCHORS[2][1]))
        area = aw * ah
        w = t3[0:1, :]
        h = t3[1:2, :]
        inter = jnp.minimum(aw, w) * jnp.minimum(ah, h)
        iou = inter / (area + w * h - inter + jnp.float32(1e-16))

        prev1 = pltpu.roll(iou, 1, 1)
        prev2 = pltpu.roll(iou, 2, 1)
        next1 = pltpu.roll(iou, _BR - 1, 1)
        next2 = pltpu.roll(iou, _BR - 2, 1)
        ciou0 = jnp.where(k0, iou, jnp.where(k1, prev1, prev2))
        ciou1 = jnp.where(k0, next1, jnp.where(k1, iou, prev1))
        ciou2 = jnp.where(k0, next2, jnp.where(k1, next1, iou))
        b0 = (ciou0 >= ciou1) & (ciou0 >= ciou2)
        b1 = jnp.logical_not(b0) & (ciou1 >= ciou2)
        b2 = jnp.logical_not(b0 | b1)
        is_best = (k0 & b0) | (k1 & b1) | ((kf >= 1.5) & b2)
        maskr = jnp.where(is_best | (iou <= jnp.float32(_IGNORE)), one, zero)

        c0 = -jnp.log(one - eps)
        s_obj = -jnp.sum(maskr * r4[3:4, :]) + jnp.float32(_C) * c0 * (
            jnp.float32(_BR) - jnp.sum(maskr))
        out_ref[...] += jnp.where(acc_lane == 4, s_obj, zero)


@jax.jit
def kernel(x, target):
    B = x.shape[0]

    out = pl.pallas_call(
        _loss_kernel,
        grid=(B, _NJ),
        in_specs=[
            pl.BlockSpec((1, _BR, _C), lambda b, j: (b, j, 0)),
            pl.BlockSpec((1, _BR, _C), lambda b, j: (b, j, 0)),
        ],
        out_specs=pl.BlockSpec((8, 128), lambda b, j: (0, 0)),
        out_shape=jax.ShapeDtypeStruct((8, 128), jnp.float32),
    )(x, target)

    s_xy = out[0, 0]
    s_wh = out[0, 1]
    s_cls = out[0, 2]
    s_m = out[0, 3]
    s_obj = out[0, 4]

    n_obj = jnp.float32(B * _N_OBJ * _C)
    return (s_xy + s_wh) / (2.0 * s_m) + s_cls / s_m + s_obj / n_obj
